# Initial kernel scaffold; baseline (speedup 1.0000x reference)
import jax
import jax.numpy as jnp
from jax.experimental import pallas as pl


def kernel(g, h, e, lb_delta, ub_delta, snorm_n, snorm_e, label, emb, W0, b0, W1, b1, W2, b2, W3, b3, mW0, mb0, mW1, mb1, mW2, mb2, rW0, rb0, rW1, rb1, rW2, rb2):
    raise NotImplementedError("write your pallas kernel here")


# speedup vs baseline: 2.9141x; 2.9141x over previous
import functools

import jax
import jax.numpy as jnp
from jax import lax
from jax.experimental import pallas as pl
from jax.experimental.pallas import tpu as pltpu
from jax.experimental.pallas import tpu_sc as plsc

N_NODES = 10000
N_EDGES = 320000
HIDDEN = 128
N_CLASSES = 10
IN_DIM = 128

NC = 2
NS = 16
NW = NC * NS

E_CHUNK = 128
E_PAD = 327680
EPT = E_PAD // NW
CHUNKS = EPT // E_CHUNK
HALF = CHUNKS // 2

ACC_ROWS = 10240
RPT = ACC_ROWS // NS
ZCH = 64
NZ = RPT // ZCH

ROW_BLK = 1000

_mesh = plsc.VectorSubcoreMesh(core_axis_name="c", subcore_axis_name="s")


@functools.partial(
    pl.kernel,
    out_type=(
        jax.ShapeDtypeStruct((ACC_ROWS, HIDDEN), jnp.float32),
        jax.ShapeDtypeStruct((NC * ACC_ROWS, HIDDEN), jnp.float32),
    ),
    mesh=_mesh,
    scratch_types=[
        pltpu.VMEM((8, 40), jnp.int32),
        pltpu.VMEM((40, HIDDEN), jnp.float32),
        pltpu.VMEM((E_CHUNK, HIDDEN), jnp.float32),
        pltpu.VMEM((HALF, E_CHUNK), jnp.int32),
        pltpu.SemaphoreType.DMA,
        pltpu.VMEM_SHARED((ACC_ROWS, HIDDEN), jnp.float32),
    ],
)
def _sc_prep(emb_hbm, h_hbm, dst_hbm, x_out, deg_out,
             h_idx, rows, work, dst_idx, sem, dacc):
    c = lax.axis_index("c")
    s = lax.axis_index("s")
    wid = s * NC + c

    pltpu.sync_copy(h_hbm.at[pl.ds(wid * 8, 8)], h_idx)

    def xstep(j, carry):
        pltpu.async_copy(emb_hbm.at[h_idx.at[j]], rows, sem).wait()
        pltpu.sync_copy(rows, x_out.at[pl.ds(wid * 320 + j * 40, 40)])
        return carry

    lax.fori_loop(0, 8, xstep, 0)

    def fill(val):
        def body(i, carry):
            for l in range(HIDDEN // 16):
                work[i, pl.ds(l * 16, 16)] = jnp.full((16,), val, jnp.float32)
            return carry
        return body

    lax.fori_loop(0, E_CHUNK, fill(0.0), 0)
    for k in range(RPT // E_CHUNK):
        pltpu.sync_copy(work, dacc.at[pl.ds(s * RPT + k * E_CHUNK, E_CHUNK)])
    lax.fori_loop(0, E_CHUNK, fill(1.0), 0)
    plsc.subcore_barrier()

    def dstep(j, carry):
        pltpu.sync_copy(work, dacc.at[dst_idx.at[j]], add=True)
        return carry

    for hf in range(2):
        pltpu.sync_copy(dst_hbm.at[pl.ds(wid * CHUNKS + hf * HALF, HALF)],
                        dst_idx)
        lax.fori_loop(0, HALF, dstep, 0)
    plsc.subcore_barrier()
    pltpu.sync_copy(dacc.at[pl.ds(s * RPT, RPT)],
                    deg_out.at[pl.ds(c * ACC_ROWS + s * RPT, RPT)])


@functools.partial(
    pl.kernel,
    out_type=jax.ShapeDtypeStruct((NC * ACC_ROWS, HIDDEN), jnp.float32),
    mesh=_mesh,
    scratch_types=[
        pltpu.VMEM((HALF, E_CHUNK), jnp.int32),
        pltpu.VMEM((HALF, E_CHUNK), jnp.int32),
        pltpu.VMEM((E_CHUNK, HIDDEN), jnp.float32),
        pltpu.VMEM((E_CHUNK, HIDDEN), jnp.float32),
        pltpu.SemaphoreType.DMA,
        pltpu.SemaphoreType.DMA,
        pltpu.VMEM_SHARED((ACC_ROWS, HIDDEN), jnp.float32),
    ],
)
def _sc_agg(x_hbm, src_hbm, dst_hbm, out_hbm,
            src_idx, dst_idx, rows0, rows1, sem0, sem1, acc):
    c = lax.axis_index("c")
    s = lax.axis_index("s")
    wid = s * NC + c

    def fill_zeros(i, carry):
        for l in range(HIDDEN // 16):
            rows0[i, pl.ds(l * 16, 16)] = jnp.zeros((16,), jnp.float32)
        return carry

    lax.fori_loop(0, E_CHUNK, fill_zeros, 0)
    for k in range(RPT // E_CHUNK):
        pltpu.sync_copy(rows0, acc.at[pl.ds(s * RPT + k * E_CHUNK, E_CHUNK)])
    plsc.subcore_barrier()

    def pair(j, carry):
        base = j * 2
        cpa = pltpu.async_copy(x_hbm.at[src_idx.at[base]], rows0, sem0)
        cpb = pltpu.async_copy(x_hbm.at[src_idx.at[base + 1]], rows1, sem1)
        cpa.wait()
        pltpu.sync_copy(rows0, acc.at[dst_idx.at[base]], add=True)
        cpb.wait()
        pltpu.sync_copy(rows1, acc.at[dst_idx.at[base + 1]], add=True)
        return carry

    for hf in range(2):
        pltpu.sync_copy(src_hbm.at[pl.ds(wid * CHUNKS + hf * HALF, HALF)],
                        src_idx)
        pltpu.sync_copy(dst_hbm.at[pl.ds(wid * CHUNKS + hf * HALF, HALF)],
                        dst_idx)
        lax.fori_loop(0, HALF // 2, pair, 0)
    plsc.subcore_barrier()

    pltpu.sync_copy(acc.at[pl.ds(s * RPT, RPT)],
                    out_hbm.at[pl.ds(c * ACC_ROWS + s * RPT, RPT)])


def _dense_body(x_ref, p0_ref, p1_ref, d0_ref, d1_ref, w_ref, b_ref, o_ref):
    agg = p0_ref[...] + p1_ref[...]
    deg = d0_ref[:, 0:1] + d1_ref[:, 0:1]
    cmean = agg * (1.0 / jnp.maximum(deg, 1.0))
    x = x_ref[...]
    w = w_ref[...]
    bundle = (
        lax.dot_general(x, w[:, :HIDDEN], (((1,), (1,)), ((), ())),
                        preferred_element_type=jnp.float32)
        + lax.dot_general(cmean, w[:, HIDDEN:], (((1,), (1,)), ((), ())),
                          preferred_element_type=jnp.float32)
        + b_ref[...]
    )
    nrm = jnp.maximum(
        jnp.sqrt(jnp.sum(bundle * bundle, axis=1, keepdims=True)), 1e-12)
    o_ref[...] = x + jnp.maximum(bundle / nrm, 0.0)


def _dense_layer(x, part0, part1, deg0, deg1, w, b):
    nblk = N_NODES // ROW_BLK
    return pl.pallas_call(
        _dense_body,
        grid=(nblk,),
        in_specs=[
            pl.BlockSpec((ROW_BLK, HIDDEN), lambda i: (i, 0)),
            pl.BlockSpec((ROW_BLK, HIDDEN), lambda i: (i, 0)),
            pl.BlockSpec((ROW_BLK, HIDDEN), lambda i: (i, 0)),
            pl.BlockSpec((ROW_BLK, HIDDEN), lambda i: (i, 0)),
            pl.BlockSpec((ROW_BLK, HIDDEN), lambda i: (i, 0)),
            pl.BlockSpec((HIDDEN, 2 * HIDDEN), lambda i: (0, 0)),
            pl.BlockSpec((1, HIDDEN), lambda i: (0, 0)),
        ],
        out_specs=pl.BlockSpec((ROW_BLK, HIDDEN), lambda i: (i, 0)),
        out_shape=jax.ShapeDtypeStruct((N_NODES, HIDDEN), jnp.float32),
    )(x, part0, part1, deg0, deg1, w, b)


def _readout_body(x_ref, lab_ref, mw0_ref, mb0_ref, mw1_ref, mb1_ref,
                  mw2_ref, mb2_ref, rw0_ref, rb0_ref, rw1_ref, rb1_ref,
                  rw2_ref, rb2_ref, lb_ref, ub_ref,
                  p_ref, g_ref, w_ref):
    x = x_ref[...]
    labp = lab_ref[...]

    y = jnp.maximum(
        lax.dot_general(x, mw0_ref[...], (((1,), (1,)), ((), ())),
                        preferred_element_type=jnp.float32) + mb0_ref[...], 0.0)
    y = jnp.maximum(
        lax.dot_general(y, mw1_ref[...], (((1,), (1,)), ((), ())),
                        preferred_element_type=jnp.float32) + mb1_ref[...], 0.0)
    p_ref[...] = (
        lax.dot_general(y, mw2_ref[...], (((1,), (1,)), ((), ())),
                        preferred_element_type=jnp.float32) + mb2_ref[...])

    hl = jnp.concatenate([x, labp], axis=1)
    z = hl + jnp.maximum(
        lax.dot_general(hl, rw0_ref[...], (((1,), (1,)), ((), ())),
                        preferred_element_type=jnp.float32) + rb0_ref[...], 0.0)
    z = z + jnp.maximum(
        lax.dot_general(z, rw1_ref[...], (((1,), (1,)), ((), ())),
                        preferred_element_type=jnp.float32) + rb1_ref[...], 0.0)
    logit = lax.dot_general(z, rw2_ref[...], (((1,), (1,)), ((), ())),
                            preferred_element_type=jnp.float32)[:, 0:1]
    w = 1.0 / (1.0 + jnp.exp(-(logit + rb2_ref[0, 0])))
    w_ref[...] = jnp.broadcast_to(w, w_ref.shape)
    wc = jnp.clip(w, lb_ref[0, 0], ub_ref[0, 0])
    g_ref[...] = (1.0 - wc) * labp + wc * (1.0 / N_CLASSES)


def _readout(x, labp, mw0p, mb0p, mw1p, mb1p, mw2p, mb2p,
             rw0p, rb0p, rw1p, rb1p, rw2p, rb2, lb, ub):
    nblk = N_NODES // ROW_BLK
    row = lambda i: (i, 0)
    const = lambda i: (0, 0)
    return pl.pallas_call(
        _readout_body,
        grid=(nblk,),
        in_specs=[
            pl.BlockSpec((ROW_BLK, HIDDEN), row),
            pl.BlockSpec((ROW_BLK, HIDDEN), row),
            pl.BlockSpec((HIDDEN, HIDDEN), const),
            pl.BlockSpec((1, HIDDEN), const),
            pl.BlockSpec((HIDDEN, HIDDEN), const),
            pl.BlockSpec((1, HIDDEN), const),
            pl.BlockSpec((HIDDEN, HIDDEN), const),
            pl.BlockSpec((1, HIDDEN), const),
            pl.BlockSpec((2 * HIDDEN, 2 * HIDDEN), const),
            pl.BlockSpec((1, 2 * HIDDEN), const),
            pl.BlockSpec((2 * HIDDEN, 2 * HIDDEN), const),
            pl.BlockSpec((1, 2 * HIDDEN), const),
            pl.BlockSpec((128, 2 * HIDDEN), const),
            pl.BlockSpec((1, 1), const),
            pl.BlockSpec((1, 1), const),
            pl.BlockSpec((1, 1), const),
        ],
        out_specs=[
            pl.BlockSpec((ROW_BLK, HIDDEN), row),
            pl.BlockSpec((ROW_BLK, HIDDEN), row),
            pl.BlockSpec((ROW_BLK, 1), row),
        ],
        out_shape=[
            jax.ShapeDtypeStruct((N_NODES, HIDDEN), jnp.float32),
            jax.ShapeDtypeStruct((N_NODES, HIDDEN), jnp.float32),
            jax.ShapeDtypeStruct((N_NODES, 1), jnp.float32),
        ],
    )(x, labp, mw0p, mb0p, mw1p, mb1p, mw2p, mb2p,
      rw0p, rb0p, rw1p, rb1p, rw2p, rb2, lb, ub)


def _pad2(a, r, c):
    return jnp.pad(a, ((0, r - a.shape[0]), (0, c - a.shape[1])))


def kernel(g, h, e, lb_delta, ub_delta, snorm_n, snorm_e, label, emb,
           W0, b0, W1, b1, W2, b2, W3, b3, mW0, mb0, mW1, mb1, mW2, mb2,
           rW0, rb0, rW1, rb1, rW2, rb2):
    src = g[0].astype(jnp.int32)
    dst = g[1].astype(jnp.int32)

    npad = E_PAD - N_EDGES
    src_p = jnp.concatenate([src, jnp.zeros((npad,), jnp.int32)])
    dst_p = jnp.concatenate([dst, jnp.full((npad,), N_NODES, jnp.int32)])
    src2d = src_p.reshape(E_PAD // E_CHUNK, E_CHUNK)
    dst2d = dst_p.reshape(E_PAD // E_CHUNK, E_CHUNK)

    h_p = jnp.concatenate([h.astype(jnp.int32),
                           jnp.zeros((ACC_ROWS - N_NODES,), jnp.int32)])
    h2d = h_p.reshape(NW * 8, 40)

    x_full, deg_parts = _sc_prep(emb, h2d, dst2d)
    x = x_full[:N_NODES]
    deg0 = deg_parts[:N_NODES]
    deg1 = deg_parts[ACC_ROWS:ACC_ROWS + N_NODES]

    for (w, b) in ((W0, b0), (W1, b1), (W2, b2), (W3, b3)):
        parts = _sc_agg(x, src2d, dst2d)
        x = _dense_layer(x, parts[:N_NODES], parts[ACC_ROWS:ACC_ROWS + N_NODES],
                         deg0, deg1, w, b.reshape(1, HIDDEN))

    labp = _pad2(label, N_NODES, HIDDEN)
    mw0p = _pad2(mW0, HIDDEN, HIDDEN)
    mb0p = _pad2(mb0.reshape(1, -1), 1, HIDDEN)
    mw1p = _pad2(mW1, HIDDEN, HIDDEN)
    mb1p = _pad2(mb1.reshape(1, -1), 1, HIDDEN)
    mw2p = _pad2(mW2, HIDDEN, HIDDEN)
    mb2p = _pad2(mb2.reshape(1, -1), 1, HIDDEN)
    rw0p = _pad2(rW0, 2 * HIDDEN, 2 * HIDDEN)
    rb0p = _pad2(rb0.reshape(1, -1), 1, 2 * HIDDEN)
    rw1p = _pad2(rW1, 2 * HIDDEN, 2 * HIDDEN)
    rb1p = _pad2(rb1.reshape(1, -1), 1, 2 * HIDDEN)
    rw2p = _pad2(rW2, 128, 2 * HIDDEN)
    rb2_2d = rb2.reshape(1, 1)
    lb2d = lb_delta.reshape(1, 1)
    ub2d = ub_delta.reshape(1, 1)

    p_pad, g_pad, w_out = _readout(
        x, labp, mw0p, mb0p, mw1p, mb1p, mw2p, mb2p,
        rw0p, rb0p, rw1p, rb1p, rw2p, rb2_2d, lb2d, ub2d)

    p = p_pad[:, :N_CLASSES]
    g_hat = g_pad[:, :N_CLASSES]
    return (p, g_hat, w_out)


# speedup vs baseline: 3.2269x; 1.1074x over previous
import functools

import jax
import jax.numpy as jnp
from jax import lax
from jax.experimental import pallas as pl
from jax.experimental.pallas import tpu as pltpu
from jax.experimental.pallas import tpu_sc as plsc

N_NODES = 10000
N_EDGES = 320000
HIDDEN = 128
N_CLASSES = 10
IN_DIM = 128

NC = 2
NS = 16
NW = NC * NS

E_CHUNK = 128
E_PAD = 327680
EPT = E_PAD // NW
CHUNKS = EPT // E_CHUNK
HALF = CHUNKS // 2

G_CHUNK = 64
G_CHUNKS = EPT // G_CHUNK
NBUF = 4
G_SLAB = 40
N_SLAB = G_CHUNKS // G_SLAB

ACC_ROWS = 10240
RPT = ACC_ROWS // NS
ZCH = 64
NZ = RPT // ZCH

ROW_BLK = 1000

_mesh = plsc.VectorSubcoreMesh(core_axis_name="c", subcore_axis_name="s")


@functools.partial(
    pl.kernel,
    out_type=(
        jax.ShapeDtypeStruct((ACC_ROWS, HIDDEN), jnp.float32),
        jax.ShapeDtypeStruct((NC * ACC_ROWS, HIDDEN), jnp.float32),
    ),
    mesh=_mesh,
    scratch_types=[
        pltpu.VMEM((8, 40), jnp.int32),
        pltpu.VMEM((40, HIDDEN), jnp.float32),
        pltpu.VMEM((E_CHUNK, HIDDEN), jnp.float32),
        pltpu.VMEM((HALF, E_CHUNK), jnp.int32),
        pltpu.SemaphoreType.DMA,
        pltpu.VMEM_SHARED((ACC_ROWS, HIDDEN), jnp.float32),
    ],
)
def _sc_prep(emb_hbm, h_hbm, dst_hbm, x_out, deg_out,
             h_idx, rows, work, dst_idx, sem, dacc):
    c = lax.axis_index("c")
    s = lax.axis_index("s")
    wid = s * NC + c

    pltpu.sync_copy(h_hbm.at[pl.ds(wid * 8, 8)], h_idx)

    def xstep(j, carry):
        pltpu.async_copy(emb_hbm.at[h_idx.at[j]], rows, sem).wait()
        pltpu.sync_copy(rows, x_out.at[pl.ds(wid * 320 + j * 40, 40)])
        return carry

    lax.fori_loop(0, 8, xstep, 0)

    def fill(val):
        def body(i, carry):
            for l in range(HIDDEN // 16):
                work[i, pl.ds(l * 16, 16)] = jnp.full((16,), val, jnp.float32)
            return carry
        return body

    lax.fori_loop(0, E_CHUNK, fill(0.0), 0)
    for k in range(RPT // E_CHUNK):
        pltpu.sync_copy(work, dacc.at[pl.ds(s * RPT + k * E_CHUNK, E_CHUNK)])
    lax.fori_loop(0, E_CHUNK, fill(1.0), 0)
    plsc.subcore_barrier()

    def dstep(j, carry):
        pltpu.sync_copy(work, dacc.at[dst_idx.at[j]], add=True)
        return carry

    for hf in range(2):
        pltpu.sync_copy(dst_hbm.at[pl.ds(wid * CHUNKS + hf * HALF, HALF)],
                        dst_idx)
        lax.fori_loop(0, HALF, dstep, 0)
    plsc.subcore_barrier()
    pltpu.sync_copy(dacc.at[pl.ds(s * RPT, RPT)],
                    deg_out.at[pl.ds(c * ACC_ROWS + s * RPT, RPT)])


@functools.partial(
    pl.kernel,
    out_type=jax.ShapeDtypeStruct((NC * ACC_ROWS, HIDDEN), jnp.float32),
    mesh=_mesh,
    scratch_types=[
        pltpu.VMEM((G_SLAB, G_CHUNK), jnp.int32),
        pltpu.VMEM((G_SLAB, G_CHUNK), jnp.int32),
        [pltpu.VMEM((G_CHUNK, HIDDEN), jnp.float32) for _ in range(NBUF)],
        [pltpu.SemaphoreType.DMA for _ in range(NBUF)],
        pltpu.VMEM_SHARED((ACC_ROWS, HIDDEN), jnp.float32),
    ],
)
def _sc_agg(x_hbm, src_hbm, dst_hbm, out_hbm,
            src_idx, dst_idx, rows, sems, acc):
    c = lax.axis_index("c")
    s = lax.axis_index("s")
    wid = s * NC + c

    def fill_zeros(i, carry):
        for l in range(HIDDEN // 16):
            rows[0][i, pl.ds(l * 16, 16)] = jnp.zeros((16,), jnp.float32)
        return carry

    lax.fori_loop(0, G_CHUNK, fill_zeros, 0)
    for k in range(RPT // G_CHUNK):
        pltpu.sync_copy(rows[0], acc.at[pl.ds(s * RPT + k * G_CHUNK, G_CHUNK)])
    plsc.subcore_barrier()

    def group(j, carry):
        base = j * NBUF
        cps = [pltpu.async_copy(x_hbm.at[src_idx.at[base + b]], rows[b],
                                sems[b]) for b in range(NBUF)]
        for b in range(NBUF):
            cps[b].wait()
            pltpu.sync_copy(rows[b], acc.at[dst_idx.at[base + b]], add=True)
        return carry

    for sl in range(N_SLAB):
        pltpu.sync_copy(src_hbm.at[pl.ds(wid * G_CHUNKS + sl * G_SLAB, G_SLAB)],
                        src_idx)
        pltpu.sync_copy(dst_hbm.at[pl.ds(wid * G_CHUNKS + sl * G_SLAB, G_SLAB)],
                        dst_idx)
        lax.fori_loop(0, G_SLAB // NBUF, group, 0)
    plsc.subcore_barrier()

    pltpu.sync_copy(acc.at[pl.ds(s * RPT, RPT)],
                    out_hbm.at[pl.ds(c * ACC_ROWS + s * RPT, RPT)])


def _dense_body(x_ref, p0_ref, p1_ref, d0_ref, d1_ref, w_ref, b_ref, o_ref):
    agg = p0_ref[...] + p1_ref[...]
    deg = d0_ref[:, 0:1] + d1_ref[:, 0:1]
    cmean = agg * (1.0 / jnp.maximum(deg, 1.0))
    x = x_ref[...]
    w = w_ref[...]
    bundle = (
        lax.dot_general(x, w[:, :HIDDEN], (((1,), (1,)), ((), ())),
                        preferred_element_type=jnp.float32)
        + lax.dot_general(cmean, w[:, HIDDEN:], (((1,), (1,)), ((), ())),
                          preferred_element_type=jnp.float32)
        + b_ref[...]
    )
    nrm = jnp.maximum(
        jnp.sqrt(jnp.sum(bundle * bundle, axis=1, keepdims=True)), 1e-12)
    o_ref[...] = x + jnp.maximum(bundle / nrm, 0.0)


def _dense_layer(x, part0, part1, deg0, deg1, w, b):
    nblk = N_NODES // ROW_BLK
    return pl.pallas_call(
        _dense_body,
        grid=(nblk,),
        in_specs=[
            pl.BlockSpec((ROW_BLK, HIDDEN), lambda i: (i, 0)),
            pl.BlockSpec((ROW_BLK, HIDDEN), lambda i: (i, 0)),
            pl.BlockSpec((ROW_BLK, HIDDEN), lambda i: (i, 0)),
            pl.BlockSpec((ROW_BLK, HIDDEN), lambda i: (i, 0)),
            pl.BlockSpec((ROW_BLK, HIDDEN), lambda i: (i, 0)),
            pl.BlockSpec((HIDDEN, 2 * HIDDEN), lambda i: (0, 0)),
            pl.BlockSpec((1, HIDDEN), lambda i: (0, 0)),
        ],
        out_specs=pl.BlockSpec((ROW_BLK, HIDDEN), lambda i: (i, 0)),
        out_shape=jax.ShapeDtypeStruct((N_NODES, HIDDEN), jnp.float32),
    )(x, part0, part1, deg0, deg1, w, b)


def _readout_body(x_ref, lab_ref, mw0_ref, mb0_ref, mw1_ref, mb1_ref,
                  mw2_ref, mb2_ref, rw0_ref, rb0_ref, rw1_ref, rb1_ref,
                  rw2_ref, rb2_ref, lb_ref, ub_ref,
                  p_ref, g_ref, w_ref):
    x = x_ref[...]
    labp = lab_ref[...]

    y = jnp.maximum(
        lax.dot_general(x, mw0_ref[...], (((1,), (1,)), ((), ())),
                        preferred_element_type=jnp.float32) + mb0_ref[...], 0.0)
    y = jnp.maximum(
        lax.dot_general(y, mw1_ref[...], (((1,), (1,)), ((), ())),
                        preferred_element_type=jnp.float32) + mb1_ref[...], 0.0)
    p_ref[...] = (
        lax.dot_general(y, mw2_ref[...], (((1,), (1,)), ((), ())),
                        preferred_element_type=jnp.float32) + mb2_ref[...])

    hl = jnp.concatenate([x, labp], axis=1)
    z = hl + jnp.maximum(
        lax.dot_general(hl, rw0_ref[...], (((1,), (1,)), ((), ())),
                        preferred_element_type=jnp.float32) + rb0_ref[...], 0.0)
    z = z + jnp.maximum(
        lax.dot_general(z, rw1_ref[...], (((1,), (1,)), ((), ())),
                        preferred_element_type=jnp.float32) + rb1_ref[...], 0.0)
    logit = lax.dot_general(z, rw2_ref[...], (((1,), (1,)), ((), ())),
                            preferred_element_type=jnp.float32)[:, 0:1]
    w = 1.0 / (1.0 + jnp.exp(-(logit + rb2_ref[0, 0])))
    w_ref[...] = jnp.broadcast_to(w, w_ref.shape)
    wc = jnp.clip(w, lb_ref[0, 0], ub_ref[0, 0])
    g_ref[...] = (1.0 - wc) * labp + wc * (1.0 / N_CLASSES)


def _readout(x, labp, mw0p, mb0p, mw1p, mb1p, mw2p, mb2p,
             rw0p, rb0p, rw1p, rb1p, rw2p, rb2, lb, ub):
    nblk = N_NODES // ROW_BLK
    row = lambda i: (i, 0)
    const = lambda i: (0, 0)
    return pl.pallas_call(
        _readout_body,
        grid=(nblk,),
        in_specs=[
            pl.BlockSpec((ROW_BLK, HIDDEN), row),
            pl.BlockSpec((ROW_BLK, HIDDEN), row),
            pl.BlockSpec((HIDDEN, HIDDEN), const),
            pl.BlockSpec((1, HIDDEN), const),
            pl.BlockSpec((HIDDEN, HIDDEN), const),
            pl.BlockSpec((1, HIDDEN), const),
            pl.BlockSpec((HIDDEN, HIDDEN), const),
            pl.BlockSpec((1, HIDDEN), const),
            pl.BlockSpec((2 * HIDDEN, 2 * HIDDEN), const),
            pl.BlockSpec((1, 2 * HIDDEN), const),
            pl.BlockSpec((2 * HIDDEN, 2 * HIDDEN), const),
            pl.BlockSpec((1, 2 * HIDDEN), const),
            pl.BlockSpec((128, 2 * HIDDEN), const),
            pl.BlockSpec((1, 1), const),
            pl.BlockSpec((1, 1), const),
            pl.BlockSpec((1, 1), const),
        ],
        out_specs=[
            pl.BlockSpec((ROW_BLK, HIDDEN), row),
            pl.BlockSpec((ROW_BLK, HIDDEN), row),
            pl.BlockSpec((ROW_BLK, 1), row),
        ],
        out_shape=[
            jax.ShapeDtypeStruct((N_NODES, HIDDEN), jnp.float32),
            jax.ShapeDtypeStruct((N_NODES, HIDDEN), jnp.float32),
            jax.ShapeDtypeStruct((N_NODES, 1), jnp.float32),
        ],
    )(x, labp, mw0p, mb0p, mw1p, mb1p, mw2p, mb2p,
      rw0p, rb0p, rw1p, rb1p, rw2p, rb2, lb, ub)


def _pad2(a, r, c):
    return jnp.pad(a, ((0, r - a.shape[0]), (0, c - a.shape[1])))


def kernel(g, h, e, lb_delta, ub_delta, snorm_n, snorm_e, label, emb,
           W0, b0, W1, b1, W2, b2, W3, b3, mW0, mb0, mW1, mb1, mW2, mb2,
           rW0, rb0, rW1, rb1, rW2, rb2):
    src = g[0].astype(jnp.int32)
    dst = g[1].astype(jnp.int32)

    npad = E_PAD - N_EDGES
    src_p = jnp.concatenate([src, jnp.zeros((npad,), jnp.int32)])
    dst_p = jnp.concatenate([dst, jnp.full((npad,), N_NODES, jnp.int32)])
    src2d = src_p.reshape(E_PAD // E_CHUNK, E_CHUNK)
    dst2d = dst_p.reshape(E_PAD // E_CHUNK, E_CHUNK)
    srcg = src_p.reshape(E_PAD // G_CHUNK, G_CHUNK)
    dstg = dst_p.reshape(E_PAD // G_CHUNK, G_CHUNK)

    h_p = jnp.concatenate([h.astype(jnp.int32),
                           jnp.zeros((ACC_ROWS - N_NODES,), jnp.int32)])
    h2d = h_p.reshape(NW * 8, 40)

    x_full, deg_parts = _sc_prep(emb, h2d, dst2d)
    x = x_full[:N_NODES]
    deg0 = deg_parts[:N_NODES]
    deg1 = deg_parts[ACC_ROWS:ACC_ROWS + N_NODES]

    for (w, b) in ((W0, b0), (W1, b1), (W2, b2), (W3, b3)):
        parts = _sc_agg(x, srcg, dstg)
        x = _dense_layer(x, parts[:N_NODES], parts[ACC_ROWS:ACC_ROWS + N_NODES],
                         deg0, deg1, w, b.reshape(1, HIDDEN))

    labp = _pad2(label, N_NODES, HIDDEN)
    mw0p = _pad2(mW0, HIDDEN, HIDDEN)
    mb0p = _pad2(mb0.reshape(1, -1), 1, HIDDEN)
    mw1p = _pad2(mW1, HIDDEN, HIDDEN)
    mb1p = _pad2(mb1.reshape(1, -1), 1, HIDDEN)
    mw2p = _pad2(mW2, HIDDEN, HIDDEN)
    mb2p = _pad2(mb2.reshape(1, -1), 1, HIDDEN)
    rw0p = _pad2(rW0, 2 * HIDDEN, 2 * HIDDEN)
    rb0p = _pad2(rb0.reshape(1, -1), 1, 2 * HIDDEN)
    rw1p = _pad2(rW1, 2 * HIDDEN, 2 * HIDDEN)
    rb1p = _pad2(rb1.reshape(1, -1), 1, 2 * HIDDEN)
    rw2p = _pad2(rW2, 128, 2 * HIDDEN)
    rb2_2d = rb2.reshape(1, 1)
    lb2d = lb_delta.reshape(1, 1)
    ub2d = ub_delta.reshape(1, 1)

    p_pad, g_pad, w_out = _readout(
        x, labp, mw0p, mb0p, mw1p, mb1p, mw2p, mb2p,
        rw0p, rb0p, rw1p, rb1p, rw2p, rb2_2d, lb2d, ub2d)

    p = p_pad[:, :N_CLASSES]
    g_hat = g_pad[:, :N_CLASSES]
    return (p, g_hat, w_out)


# speedup vs baseline: 3.6560x; 1.1330x over previous
import functools

import jax
import jax.numpy as jnp
from jax import lax
from jax.experimental import pallas as pl
from jax.experimental.pallas import tpu as pltpu
from jax.experimental.pallas import tpu_sc as plsc

N_NODES = 10000
N_EDGES = 320000
HIDDEN = 128
N_CLASSES = 10
IN_DIM = 128

NC = 2
NS = 16
NW = NC * NS

E_CHUNK = 128
E_PAD = 327680
EPT = E_PAD // NW
CHUNKS = EPT // E_CHUNK
HALF = CHUNKS // 2

G_CHUNK = 64
G_CHUNKS = EPT // G_CHUNK
NBUF = 4
G_SLAB = 40
N_SLAB = G_CHUNKS // G_SLAB

G_TOTAL = E_PAD // G_CHUNK
N0C = 240
N1C = 80

ACC_ROWS = 10240
RPT = ACC_ROWS // NS
ZCH = 64
NZ = RPT // ZCH

ROW_BLK = 1000

_mesh = plsc.VectorSubcoreMesh(core_axis_name="c", subcore_axis_name="s")


@functools.partial(
    pl.kernel,
    out_type=(
        jax.ShapeDtypeStruct((ACC_ROWS, HIDDEN), jnp.float32),
        jax.ShapeDtypeStruct((NC * ACC_ROWS, HIDDEN), jnp.float32),
    ),
    mesh=_mesh,
    scratch_types=[
        pltpu.VMEM((8, 40), jnp.int32),
        pltpu.VMEM((40, HIDDEN), jnp.float32),
        pltpu.VMEM((E_CHUNK, HIDDEN), jnp.float32),
        pltpu.VMEM((HALF, E_CHUNK), jnp.int32),
        pltpu.SemaphoreType.DMA,
        pltpu.VMEM_SHARED((ACC_ROWS, HIDDEN), jnp.float32),
    ],
)
def _sc_prep(emb_hbm, h_hbm, dst_hbm, x_out, deg_out,
             h_idx, rows, work, dst_idx, sem, dacc):
    c = lax.axis_index("c")
    s = lax.axis_index("s")
    wid = s * NC + c

    pltpu.sync_copy(h_hbm.at[pl.ds(wid * 8, 8)], h_idx)

    def xstep(j, carry):
        pltpu.async_copy(emb_hbm.at[h_idx.at[j]], rows, sem).wait()
        pltpu.sync_copy(rows, x_out.at[pl.ds(wid * 320 + j * 40, 40)])
        return carry

    lax.fori_loop(0, 8, xstep, 0)

    def fill(val):
        def body(i, carry):
            for l in range(HIDDEN // 16):
                work[i, pl.ds(l * 16, 16)] = jnp.full((16,), val, jnp.float32)
            return carry
        return body

    lax.fori_loop(0, E_CHUNK, fill(0.0), 0)
    for k in range(RPT // E_CHUNK):
        pltpu.sync_copy(work, dacc.at[pl.ds(s * RPT + k * E_CHUNK, E_CHUNK)])
    lax.fori_loop(0, E_CHUNK, fill(1.0), 0)
    plsc.subcore_barrier()

    def dstep(j, carry):
        pltpu.sync_copy(work, dacc.at[dst_idx.at[j]], add=True)
        return carry

    for hf in range(2):
        pltpu.sync_copy(dst_hbm.at[pl.ds(wid * CHUNKS + hf * HALF, HALF)],
                        dst_idx)
        lax.fori_loop(0, HALF, dstep, 0)
    plsc.subcore_barrier()
    pltpu.sync_copy(dacc.at[pl.ds(s * RPT, RPT)],
                    deg_out.at[pl.ds(c * ACC_ROWS + s * RPT, RPT)])


@functools.partial(
    pl.kernel,
    out_type=jax.ShapeDtypeStruct((NC * ACC_ROWS, HIDDEN), jnp.float32),
    mesh=_mesh,
    scratch_types=[
        pltpu.VMEM((G_SLAB, G_CHUNK), jnp.int32),
        pltpu.VMEM((G_SLAB, G_CHUNK), jnp.int32),
        [pltpu.VMEM((G_CHUNK, HIDDEN), jnp.float32) for _ in range(NBUF)],
        [pltpu.SemaphoreType.DMA for _ in range(NBUF)],
        pltpu.VMEM_SHARED((ACC_ROWS, HIDDEN), jnp.float32),
    ],
)
def _sc_agg(x_hbm, src_hbm, dst_hbm, out_hbm,
            src_idx, dst_idx, rows, sems, acc):
    c = lax.axis_index("c")
    s = lax.axis_index("s")
    wid = s * NC + c

    def fill_zeros(i, carry):
        for l in range(HIDDEN // 16):
            rows[0][i, pl.ds(l * 16, 16)] = jnp.zeros((16,), jnp.float32)
        return carry

    lax.fori_loop(0, G_CHUNK, fill_zeros, 0)
    for k in range(RPT // G_CHUNK):
        pltpu.sync_copy(rows[0], acc.at[pl.ds(s * RPT + k * G_CHUNK, G_CHUNK)])
    plsc.subcore_barrier()

    def group(j, carry):
        base = j * NBUF
        cps = [pltpu.async_copy(x_hbm.at[src_idx.at[base + b]], rows[b],
                                sems[b]) for b in range(NBUF)]
        for b in range(NBUF):
            cps[b].wait()
            pltpu.sync_copy(rows[b], acc.at[dst_idx.at[base + b]], add=True)
        return carry

    def run(chunk_base, nslab):
        for sl in range(nslab):
            pltpu.sync_copy(
                src_hbm.at[pl.ds(chunk_base + sl * G_SLAB, G_SLAB)], src_idx)
            pltpu.sync_copy(
                dst_hbm.at[pl.ds(chunk_base + sl * G_SLAB, G_SLAB)], dst_idx)
            lax.fori_loop(0, G_SLAB // NBUF, group, 0)

    @pl.when(c == 0)
    def _():
        run(s * N0C, N0C // G_SLAB)

    @pl.when(c == 1)
    def _():
        run(NS * N0C + s * N1C, N1C // G_SLAB)

    plsc.subcore_barrier()

    pltpu.sync_copy(acc.at[pl.ds(s * RPT, RPT)],
                    out_hbm.at[pl.ds(c * ACC_ROWS + s * RPT, RPT)])


def _dense_body(x_ref, p0_ref, p1_ref, d0_ref, d1_ref, w_ref, b_ref, o_ref):
    agg = p0_ref[...] + p1_ref[...]
    deg = d0_ref[:, 0:1] + d1_ref[:, 0:1]
    cmean = agg * (1.0 / jnp.maximum(deg, 1.0))
    x = x_ref[...]
    w = w_ref[...]
    bundle = (
        lax.dot_general(x, w[:, :HIDDEN], (((1,), (1,)), ((), ())),
                        preferred_element_type=jnp.float32)
        + lax.dot_general(cmean, w[:, HIDDEN:], (((1,), (1,)), ((), ())),
                          preferred_element_type=jnp.float32)
        + b_ref[...]
    )
    nrm = jnp.maximum(
        jnp.sqrt(jnp.sum(bundle * bundle, axis=1, keepdims=True)), 1e-12)
    o_ref[...] = x + jnp.maximum(bundle / nrm, 0.0)


def _dense_layer(x, part0, part1, deg0, deg1, w, b):
    nblk = N_NODES // ROW_BLK
    return pl.pallas_call(
        _dense_body,
        grid=(nblk,),
        in_specs=[
            pl.BlockSpec((ROW_BLK, HIDDEN), lambda i: (i, 0)),
            pl.BlockSpec((ROW_BLK, HIDDEN), lambda i: (i, 0)),
            pl.BlockSpec((ROW_BLK, HIDDEN), lambda i: (i, 0)),
            pl.BlockSpec((ROW_BLK, HIDDEN), lambda i: (i, 0)),
            pl.BlockSpec((ROW_BLK, HIDDEN), lambda i: (i, 0)),
            pl.BlockSpec((HIDDEN, 2 * HIDDEN), lambda i: (0, 0)),
            pl.BlockSpec((1, HIDDEN), lambda i: (0, 0)),
        ],
        out_specs=pl.BlockSpec((ROW_BLK, HIDDEN), lambda i: (i, 0)),
        out_shape=jax.ShapeDtypeStruct((N_NODES, HIDDEN), jnp.float32),
    )(x, part0, part1, deg0, deg1, w, b)


def _readout_body(x_ref, lab_ref, mw0_ref, mb0_ref, mw1_ref, mb1_ref,
                  mw2_ref, mb2_ref, rw0_ref, rb0_ref, rw1_ref, rb1_ref,
                  rw2_ref, rb2_ref, lb_ref, ub_ref,
                  p_ref, g_ref, w_ref):
    x = x_ref[...]
    labp = lab_ref[...]

    y = jnp.maximum(
        lax.dot_general(x, mw0_ref[...], (((1,), (1,)), ((), ())),
                        preferred_element_type=jnp.float32) + mb0_ref[...], 0.0)
    y = jnp.maximum(
        lax.dot_general(y, mw1_ref[...], (((1,), (1,)), ((), ())),
                        preferred_element_type=jnp.float32) + mb1_ref[...], 0.0)
    p_ref[...] = (
        lax.dot_general(y, mw2_ref[...], (((1,), (1,)), ((), ())),
                        preferred_element_type=jnp.float32) + mb2_ref[...])

    hl = jnp.concatenate([x, labp], axis=1)
    z = hl + jnp.maximum(
        lax.dot_general(hl, rw0_ref[...], (((1,), (1,)), ((), ())),
                        preferred_element_type=jnp.float32) + rb0_ref[...], 0.0)
    z = z + jnp.maximum(
        lax.dot_general(z, rw1_ref[...], (((1,), (1,)), ((), ())),
                        preferred_element_type=jnp.float32) + rb1_ref[...], 0.0)
    logit = lax.dot_general(z, rw2_ref[...], (((1,), (1,)), ((), ())),
                            preferred_element_type=jnp.float32)[:, 0:1]
    w = 1.0 / (1.0 + jnp.exp(-(logit + rb2_ref[0, 0])))
    w_ref[...] = jnp.broadcast_to(w, w_ref.shape)
    wc = jnp.clip(w, lb_ref[0, 0], ub_ref[0, 0])
    g_ref[...] = (1.0 - wc) * labp + wc * (1.0 / N_CLASSES)


def _readout(x, labp, mw0p, mb0p, mw1p, mb1p, mw2p, mb2p,
             rw0p, rb0p, rw1p, rb1p, rw2p, rb2, lb, ub):
    nblk = N_NODES // ROW_BLK
    row = lambda i: (i, 0)
    const = lambda i: (0, 0)
    return pl.pallas_call(
        _readout_body,
        grid=(nblk,),
        in_specs=[
            pl.BlockSpec((ROW_BLK, HIDDEN), row),
            pl.BlockSpec((ROW_BLK, HIDDEN), row),
            pl.BlockSpec((HIDDEN, HIDDEN), const),
            pl.BlockSpec((1, HIDDEN), const),
            pl.BlockSpec((HIDDEN, HIDDEN), const),
            pl.BlockSpec((1, HIDDEN), const),
            pl.BlockSpec((HIDDEN, HIDDEN), const),
            pl.BlockSpec((1, HIDDEN), const),
            pl.BlockSpec((2 * HIDDEN, 2 * HIDDEN), const),
            pl.BlockSpec((1, 2 * HIDDEN), const),
            pl.BlockSpec((2 * HIDDEN, 2 * HIDDEN), const),
            pl.BlockSpec((1, 2 * HIDDEN), const),
            pl.BlockSpec((128, 2 * HIDDEN), const),
            pl.BlockSpec((1, 1), const),
            pl.BlockSpec((1, 1), const),
            pl.BlockSpec((1, 1), const),
        ],
        out_specs=[
            pl.BlockSpec((ROW_BLK, HIDDEN), row),
            pl.BlockSpec((ROW_BLK, HIDDEN), row),
            pl.BlockSpec((ROW_BLK, 1), row),
        ],
        out_shape=[
            jax.ShapeDtypeStruct((N_NODES, HIDDEN), jnp.float32),
            jax.ShapeDtypeStruct((N_NODES, HIDDEN), jnp.float32),
            jax.ShapeDtypeStruct((N_NODES, 1), jnp.float32),
        ],
    )(x, labp, mw0p, mb0p, mw1p, mb1p, mw2p, mb2p,
      rw0p, rb0p, rw1p, rb1p, rw2p, rb2, lb, ub)


def _pad2(a, r, c):
    return jnp.pad(a, ((0, r - a.shape[0]), (0, c - a.shape[1])))


def kernel(g, h, e, lb_delta, ub_delta, snorm_n, snorm_e, label, emb,
           W0, b0, W1, b1, W2, b2, W3, b3, mW0, mb0, mW1, mb1, mW2, mb2,
           rW0, rb0, rW1, rb1, rW2, rb2):
    src = g[0].astype(jnp.int32)
    dst = g[1].astype(jnp.int32)

    npad = E_PAD - N_EDGES
    src_p = jnp.concatenate([src, jnp.zeros((npad,), jnp.int32)])
    dst_p = jnp.concatenate([dst, jnp.full((npad,), N_NODES, jnp.int32)])
    src2d = src_p.reshape(E_PAD // E_CHUNK, E_CHUNK)
    dst2d = dst_p.reshape(E_PAD // E_CHUNK, E_CHUNK)
    srcg = src_p.reshape(E_PAD // G_CHUNK, G_CHUNK)
    dstg = dst_p.reshape(E_PAD // G_CHUNK, G_CHUNK)

    h_p = jnp.concatenate([h.astype(jnp.int32),
                           jnp.zeros((ACC_ROWS - N_NODES,), jnp.int32)])
    h2d = h_p.reshape(NW * 8, 40)

    x_full, deg_parts = _sc_prep(emb, h2d, dst2d)
    x = x_full[:N_NODES]
    deg0 = deg_parts[:N_NODES]
    deg1 = deg_parts[ACC_ROWS:ACC_ROWS + N_NODES]

    for (w, b) in ((W0, b0), (W1, b1), (W2, b2), (W3, b3)):
        parts = _sc_agg(x, srcg, dstg)
        x = _dense_layer(x, parts[:N_NODES], parts[ACC_ROWS:ACC_ROWS + N_NODES],
                         deg0, deg1, w, b.reshape(1, HIDDEN))

    labp = _pad2(label, N_NODES, HIDDEN)
    mw0p = _pad2(mW0, HIDDEN, HIDDEN)
    mb0p = _pad2(mb0.reshape(1, -1), 1, HIDDEN)
    mw1p = _pad2(mW1, HIDDEN, HIDDEN)
    mb1p = _pad2(mb1.reshape(1, -1), 1, HIDDEN)
    mw2p = _pad2(mW2, HIDDEN, HIDDEN)
    mb2p = _pad2(mb2.reshape(1, -1), 1, HIDDEN)
    rw0p = _pad2(rW0, 2 * HIDDEN, 2 * HIDDEN)
    rb0p = _pad2(rb0.reshape(1, -1), 1, 2 * HIDDEN)
    rw1p = _pad2(rW1, 2 * HIDDEN, 2 * HIDDEN)
    rb1p = _pad2(rb1.reshape(1, -1), 1, 2 * HIDDEN)
    rw2p = _pad2(rW2, 128, 2 * HIDDEN)
    rb2_2d = rb2.reshape(1, 1)
    lb2d = lb_delta.reshape(1, 1)
    ub2d = ub_delta.reshape(1, 1)

    p_pad, g_pad, w_out = _readout(
        x, labp, mw0p, mb0p, mw1p, mb1p, mw2p, mb2p,
        rw0p, rb0p, rw1p, rb1p, rw2p, rb2_2d, lb2d, ub2d)

    p = p_pad[:, :N_CLASSES]
    g_hat = g_pad[:, :N_CLASSES]
    return (p, g_hat, w_out)


# speedup vs baseline: 3.8533x; 1.0540x over previous
import functools

import jax
import jax.numpy as jnp
from jax import lax
from jax.experimental import pallas as pl
from jax.experimental.pallas import tpu as pltpu
from jax.experimental.pallas import tpu_sc as plsc

N_NODES = 10000
N_EDGES = 320000
HIDDEN = 128
N_CLASSES = 10
IN_DIM = 128

NC = 2
NS = 16
NW = NC * NS

E_CHUNK = 128
E_PAD = 327680
EPT = E_PAD // NW
CHUNKS = EPT // E_CHUNK
HALF = CHUNKS // 2

G_CHUNK = 64
G_CHUNKS = EPT // G_CHUNK
NBUF = 4
G_SLAB = 40
N_SLAB = G_CHUNKS // G_SLAB

G_TOTAL = E_PAD // G_CHUNK
N0C = 280
N1C = 40

ACC_ROWS = 10240
RPT = ACC_ROWS // NS
ZCH = 64
NZ = RPT // ZCH

ROW_BLK = 1000

_mesh = plsc.VectorSubcoreMesh(core_axis_name="c", subcore_axis_name="s")


@functools.partial(
    pl.kernel,
    out_type=(
        jax.ShapeDtypeStruct((ACC_ROWS, HIDDEN), jnp.float32),
        jax.ShapeDtypeStruct((NC * ACC_ROWS, HIDDEN), jnp.float32),
    ),
    mesh=_mesh,
    scratch_types=[
        pltpu.VMEM((8, 40), jnp.int32),
        pltpu.VMEM((40, HIDDEN), jnp.float32),
        pltpu.VMEM((E_CHUNK, HIDDEN), jnp.float32),
        pltpu.VMEM((HALF, E_CHUNK), jnp.int32),
        pltpu.SemaphoreType.DMA,
        pltpu.VMEM_SHARED((ACC_ROWS, HIDDEN), jnp.float32),
    ],
)
def _sc_prep(emb_hbm, h_hbm, dst_hbm, x_out, deg_out,
             h_idx, rows, work, dst_idx, sem, dacc):
    c = lax.axis_index("c")
    s = lax.axis_index("s")
    wid = s * NC + c

    pltpu.sync_copy(h_hbm.at[pl.ds(wid * 8, 8)], h_idx)

    def xstep(j, carry):
        pltpu.async_copy(emb_hbm.at[h_idx.at[j]], rows, sem).wait()
        pltpu.sync_copy(rows, x_out.at[pl.ds(wid * 320 + j * 40, 40)])
        return carry

    lax.fori_loop(0, 8, xstep, 0)

    def fill(val):
        def body(i, carry):
            for l in range(HIDDEN // 16):
                work[i, pl.ds(l * 16, 16)] = jnp.full((16,), val, jnp.float32)
            return carry
        return body

    lax.fori_loop(0, E_CHUNK, fill(0.0), 0)
    for k in range(RPT // E_CHUNK):
        pltpu.sync_copy(work, dacc.at[pl.ds(s * RPT + k * E_CHUNK, E_CHUNK)])
    lax.fori_loop(0, E_CHUNK, fill(1.0), 0)
    plsc.subcore_barrier()

    def dstep(j, carry):
        pltpu.sync_copy(work, dacc.at[dst_idx.at[j]], add=True)
        return carry

    for hf in range(2):
        pltpu.sync_copy(dst_hbm.at[pl.ds(wid * CHUNKS + hf * HALF, HALF)],
                        dst_idx)
        lax.fori_loop(0, HALF, dstep, 0)
    plsc.subcore_barrier()
    pltpu.sync_copy(dacc.at[pl.ds(s * RPT, RPT)],
                    deg_out.at[pl.ds(c * ACC_ROWS + s * RPT, RPT)])


@functools.partial(
    pl.kernel,
    out_type=jax.ShapeDtypeStruct((NC * ACC_ROWS, HIDDEN), jnp.float32),
    mesh=_mesh,
    scratch_types=[
        pltpu.VMEM((G_SLAB, G_CHUNK), jnp.int32),
        pltpu.VMEM((G_SLAB, G_CHUNK), jnp.int32),
        [pltpu.VMEM((G_CHUNK, HIDDEN), jnp.float32) for _ in range(NBUF)],
        [pltpu.SemaphoreType.DMA for _ in range(NBUF)],
        pltpu.VMEM_SHARED((ACC_ROWS, HIDDEN), jnp.float32),
    ],
)
def _sc_agg(x_hbm, src_hbm, dst_hbm, out_hbm,
            src_idx, dst_idx, rows, sems, acc):
    c = lax.axis_index("c")
    s = lax.axis_index("s")
    wid = s * NC + c

    def fill_zeros(i, carry):
        for l in range(HIDDEN // 16):
            rows[0][i, pl.ds(l * 16, 16)] = jnp.zeros((16,), jnp.float32)
        return carry

    lax.fori_loop(0, G_CHUNK, fill_zeros, 0)
    for k in range(RPT // G_CHUNK):
        pltpu.sync_copy(rows[0], acc.at[pl.ds(s * RPT + k * G_CHUNK, G_CHUNK)])
    plsc.subcore_barrier()

    def round_(j, carry):
        for b in range(NBUF):
            ch = j * NBUF + b
            pltpu.make_async_copy(x_hbm.at[src_idx.at[ch]], rows[b],
                                  sems[b]).wait()
            pltpu.sync_copy(rows[b], acc.at[dst_idx.at[ch]], add=True)

            @pl.when(ch + NBUF < G_SLAB)
            def _():
                pltpu.async_copy(x_hbm.at[src_idx.at[ch + NBUF]], rows[b],
                                 sems[b])
        return carry

    def run(chunk_base, nslab):
        for sl in range(nslab):
            pltpu.sync_copy(
                src_hbm.at[pl.ds(chunk_base + sl * G_SLAB, G_SLAB)], src_idx)
            pltpu.sync_copy(
                dst_hbm.at[pl.ds(chunk_base + sl * G_SLAB, G_SLAB)], dst_idx)
            for b in range(NBUF):
                pltpu.async_copy(x_hbm.at[src_idx.at[b]], rows[b], sems[b])
            lax.fori_loop(0, G_SLAB // NBUF, round_, 0)

    @pl.when(c == 0)
    def _():
        run(s * N0C, N0C // G_SLAB)

    @pl.when(c == 1)
    def _():
        run(NS * N0C + s * N1C, N1C // G_SLAB)

    plsc.subcore_barrier()

    pltpu.sync_copy(acc.at[pl.ds(s * RPT, RPT)],
                    out_hbm.at[pl.ds(c * ACC_ROWS + s * RPT, RPT)])


def _dense_body(x_ref, p0_ref, p1_ref, d0_ref, d1_ref, w_ref, b_ref, o_ref):
    agg = p0_ref[...] + p1_ref[...]
    deg = d0_ref[:, 0:1] + d1_ref[:, 0:1]
    cmean = agg * (1.0 / jnp.maximum(deg, 1.0))
    x = x_ref[...]
    w = w_ref[...]
    bundle = (
        lax.dot_general(x, w[:, :HIDDEN], (((1,), (1,)), ((), ())),
                        preferred_element_type=jnp.float32)
        + lax.dot_general(cmean, w[:, HIDDEN:], (((1,), (1,)), ((), ())),
                          preferred_element_type=jnp.float32)
        + b_ref[...]
    )
    nrm = jnp.maximum(
        jnp.sqrt(jnp.sum(bundle * bundle, axis=1, keepdims=True)), 1e-12)
    o_ref[...] = x + jnp.maximum(bundle / nrm, 0.0)


def _dense_layer(x, part0, part1, deg0, deg1, w, b):
    nblk = N_NODES // ROW_BLK
    return pl.pallas_call(
        _dense_body,
        grid=(nblk,),
        in_specs=[
            pl.BlockSpec((ROW_BLK, HIDDEN), lambda i: (i, 0)),
            pl.BlockSpec((ROW_BLK, HIDDEN), lambda i: (i, 0)),
            pl.BlockSpec((ROW_BLK, HIDDEN), lambda i: (i, 0)),
            pl.BlockSpec((ROW_BLK, HIDDEN), lambda i: (i, 0)),
            pl.BlockSpec((ROW_BLK, HIDDEN), lambda i: (i, 0)),
            pl.BlockSpec((HIDDEN, 2 * HIDDEN), lambda i: (0, 0)),
            pl.BlockSpec((1, HIDDEN), lambda i: (0, 0)),
        ],
        out_specs=pl.BlockSpec((ROW_BLK, HIDDEN), lambda i: (i, 0)),
        out_shape=jax.ShapeDtypeStruct((N_NODES, HIDDEN), jnp.float32),
    )(x, part0, part1, deg0, deg1, w, b)


def _readout_body(x_ref, lab_ref, mw0_ref, mb0_ref, mw1_ref, mb1_ref,
                  mw2_ref, mb2_ref, rw0_ref, rb0_ref, rw1_ref, rb1_ref,
                  rw2_ref, rb2_ref, lb_ref, ub_ref,
                  p_ref, g_ref, w_ref):
    x = x_ref[...]
    labp = lab_ref[...]

    y = jnp.maximum(
        lax.dot_general(x, mw0_ref[...], (((1,), (1,)), ((), ())),
                        preferred_element_type=jnp.float32) + mb0_ref[...], 0.0)
    y = jnp.maximum(
        lax.dot_general(y, mw1_ref[...], (((1,), (1,)), ((), ())),
                        preferred_element_type=jnp.float32) + mb1_ref[...], 0.0)
    p_ref[...] = (
        lax.dot_general(y, mw2_ref[...], (((1,), (1,)), ((), ())),
                        preferred_element_type=jnp.float32) + mb2_ref[...])

    hl = jnp.concatenate([x, labp], axis=1)
    z = hl + jnp.maximum(
        lax.dot_general(hl, rw0_ref[...], (((1,), (1,)), ((), ())),
                        preferred_element_type=jnp.float32) + rb0_ref[...], 0.0)
    z = z + jnp.maximum(
        lax.dot_general(z, rw1_ref[...], (((1,), (1,)), ((), ())),
                        preferred_element_type=jnp.float32) + rb1_ref[...], 0.0)
    logit = lax.dot_general(z, rw2_ref[...], (((1,), (1,)), ((), ())),
                            preferred_element_type=jnp.float32)[:, 0:1]
    w = 1.0 / (1.0 + jnp.exp(-(logit + rb2_ref[0, 0])))
    w_ref[...] = jnp.broadcast_to(w, w_ref.shape)
    wc = jnp.clip(w, lb_ref[0, 0], ub_ref[0, 0])
    g_ref[...] = (1.0 - wc) * labp + wc * (1.0 / N_CLASSES)


def _readout(x, labp, mw0p, mb0p, mw1p, mb1p, mw2p, mb2p,
             rw0p, rb0p, rw1p, rb1p, rw2p, rb2, lb, ub):
    nblk = N_NODES // ROW_BLK
    row = lambda i: (i, 0)
    const = lambda i: (0, 0)
    return pl.pallas_call(
        _readout_body,
        grid=(nblk,),
        in_specs=[
            pl.BlockSpec((ROW_BLK, HIDDEN), row),
            pl.BlockSpec((ROW_BLK, HIDDEN), row),
            pl.BlockSpec((HIDDEN, HIDDEN), const),
            pl.BlockSpec((1, HIDDEN), const),
            pl.BlockSpec((HIDDEN, HIDDEN), const),
            pl.BlockSpec((1, HIDDEN), const),
            pl.BlockSpec((HIDDEN, HIDDEN), const),
            pl.BlockSpec((1, HIDDEN), const),
            pl.BlockSpec((2 * HIDDEN, 2 * HIDDEN), const),
            pl.BlockSpec((1, 2 * HIDDEN), const),
            pl.BlockSpec((2 * HIDDEN, 2 * HIDDEN), const),
            pl.BlockSpec((1, 2 * HIDDEN), const),
            pl.BlockSpec((128, 2 * HIDDEN), const),
            pl.BlockSpec((1, 1), const),
            pl.BlockSpec((1, 1), const),
            pl.BlockSpec((1, 1), const),
        ],
        out_specs=[
            pl.BlockSpec((ROW_BLK, HIDDEN), row),
            pl.BlockSpec((ROW_BLK, HIDDEN), row),
            pl.BlockSpec((ROW_BLK, 1), row),
        ],
        out_shape=[
            jax.ShapeDtypeStruct((N_NODES, HIDDEN), jnp.float32),
            jax.ShapeDtypeStruct((N_NODES, HIDDEN), jnp.float32),
            jax.ShapeDtypeStruct((N_NODES, 1), jnp.float32),
        ],
    )(x, labp, mw0p, mb0p, mw1p, mb1p, mw2p, mb2p,
      rw0p, rb0p, rw1p, rb1p, rw2p, rb2, lb, ub)


def _pad2(a, r, c):
    return jnp.pad(a, ((0, r - a.shape[0]), (0, c - a.shape[1])))


def kernel(g, h, e, lb_delta, ub_delta, snorm_n, snorm_e, label, emb,
           W0, b0, W1, b1, W2, b2, W3, b3, mW0, mb0, mW1, mb1, mW2, mb2,
           rW0, rb0, rW1, rb1, rW2, rb2):
    src = g[0].astype(jnp.int32)
    dst = g[1].astype(jnp.int32)

    npad = E_PAD - N_EDGES
    src_p = jnp.concatenate([src, jnp.zeros((npad,), jnp.int32)])
    dst_p = jnp.concatenate([dst, jnp.full((npad,), N_NODES, jnp.int32)])
    src2d = src_p.reshape(E_PAD // E_CHUNK, E_CHUNK)
    dst2d = dst_p.reshape(E_PAD // E_CHUNK, E_CHUNK)
    srcg = src_p.reshape(E_PAD // G_CHUNK, G_CHUNK)
    dstg = dst_p.reshape(E_PAD // G_CHUNK, G_CHUNK)

    h_p = jnp.concatenate([h.astype(jnp.int32),
                           jnp.zeros((ACC_ROWS - N_NODES,), jnp.int32)])
    h2d = h_p.reshape(NW * 8, 40)

    x_full, deg_parts = _sc_prep(emb, h2d, dst2d)
    x = x_full[:N_NODES]
    deg0 = deg_parts[:N_NODES]
    deg1 = deg_parts[ACC_ROWS:ACC_ROWS + N_NODES]

    for (w, b) in ((W0, b0), (W1, b1), (W2, b2), (W3, b3)):
        parts = _sc_agg(x, srcg, dstg)
        x = _dense_layer(x, parts[:N_NODES], parts[ACC_ROWS:ACC_ROWS + N_NODES],
                         deg0, deg1, w, b.reshape(1, HIDDEN))

    labp = _pad2(label, N_NODES, HIDDEN)
    mw0p = _pad2(mW0, HIDDEN, HIDDEN)
    mb0p = _pad2(mb0.reshape(1, -1), 1, HIDDEN)
    mw1p = _pad2(mW1, HIDDEN, HIDDEN)
    mb1p = _pad2(mb1.reshape(1, -1), 1, HIDDEN)
    mw2p = _pad2(mW2, HIDDEN, HIDDEN)
    mb2p = _pad2(mb2.reshape(1, -1), 1, HIDDEN)
    rw0p = _pad2(rW0, 2 * HIDDEN, 2 * HIDDEN)
    rb0p = _pad2(rb0.reshape(1, -1), 1, 2 * HIDDEN)
    rw1p = _pad2(rW1, 2 * HIDDEN, 2 * HIDDEN)
    rb1p = _pad2(rb1.reshape(1, -1), 1, 2 * HIDDEN)
    rw2p = _pad2(rW2, 128, 2 * HIDDEN)
    rb2_2d = rb2.reshape(1, 1)
    lb2d = lb_delta.reshape(1, 1)
    ub2d = ub_delta.reshape(1, 1)

    p_pad, g_pad, w_out = _readout(
        x, labp, mw0p, mb0p, mw1p, mb1p, mw2p, mb2p,
        rw0p, rb0p, rw1p, rb1p, rw2p, rb2_2d, lb2d, ub2d)

    p = p_pad[:, :N_CLASSES]
    g_hat = g_pad[:, :N_CLASSES]
    return (p, g_hat, w_out)
